# baseline (speedup 1.0000x reference)
import functools

import jax
import jax.numpy as jnp
from jax import lax
from jax.experimental import pallas as pl
from jax.experimental.pallas import tpu as pltpu
from jax.experimental.pallas import tpu_sc as plsc

B = 4096
V = 100000
D = 1536
H = 512

NC = 2
NS = 16
NW = NC * NS
CH = 32

G = 2
BCH = B // G
ROWS_PER_W = BCH // NW
NCHUNK = ROWS_PER_W // CH

BM = 512


def _sc_gather_body(idx_hbm, jd_tab, cv_tab, jd_out, cv_out,
                    idx_v, buf0, buf1, sem0, sem1):
  cid = lax.axis_index("c")
  sid = lax.axis_index("s")
  wid = sid * NC + cid
  pltpu.sync_copy(idx_hbm.at[wid], idx_v)
  base = wid * ROWS_PER_W
  bufs = (buf0, buf1)
  sems = (sem0, sem1)
  work = [(t, ch) for t in range(2) for ch in range(NCHUNK)]
  tabs = (jd_tab, cv_tab)
  outs = (jd_out, cv_out)
  copies = [None, None]
  for k, (t, ch) in enumerate(work):
    p = k % 2
    copies[p] = pltpu.async_copy(tabs[t].at[idx_v.at[t, ch]], bufs[p], sems[p])
    if k >= 1:
      pt, pch = work[k - 1]
      copies[(k - 1) % 2].wait()
      pltpu.sync_copy(bufs[(k - 1) % 2],
                      outs[pt].at[pl.ds(base + pch * CH, CH)])
  lt, lch = work[-1]
  copies[(len(work) - 1) % 2].wait()
  pltpu.sync_copy(bufs[(len(work) - 1) % 2],
                  outs[lt].at[pl.ds(base + lch * CH, CH)])


def _sc_gather(idx, jd_table, cv_table):
  mesh = plsc.VectorSubcoreMesh(core_axis_name="c", subcore_axis_name="s")
  return pl.kernel(
      _sc_gather_body,
      mesh=mesh,
      out_type=[
          jax.ShapeDtypeStruct((BCH, D), jnp.float32),
          jax.ShapeDtypeStruct((BCH, D), jnp.float32),
      ],
      scratch_types=[
          pltpu.VMEM((2, NCHUNK, CH), jnp.int32),
          pltpu.VMEM((CH, D), jnp.float32),
          pltpu.VMEM((CH, D), jnp.float32),
          pltpu.SemaphoreType.DMA,
          pltpu.SemaphoreType.DMA,
      ],
  )(idx, jd_table, cv_table)


def _mlp_body(jd_ref, cv_ref, wj_ref, wc_ref, bc_ref, w1_ref, b1_ref,
              w2_ref, b2_ref, out_ref):
  x = (jnp.dot(jd_ref[...], wj_ref[...], preferred_element_type=jnp.float32)
       + jnp.dot(cv_ref[...], wc_ref[...], preferred_element_type=jnp.float32)
       + bc_ref[...])
  x = jnp.where(x >= 0, x, 0.01 * x)
  x = jnp.dot(x, w1_ref[...], preferred_element_type=jnp.float32) + b1_ref[...]
  x = jnp.where(x >= 0, x, 0.01 * x)
  out_ref[...] = (
      jnp.dot(x, w2_ref[...], preferred_element_type=jnp.float32) + b2_ref[...])


def _mlp(jd_e, cv_e, wj, wc, bc, w1, b1, w2, b2):
  return pl.pallas_call(
      _mlp_body,
      grid=(BCH // BM,),
      in_specs=[
          pl.BlockSpec((BM, D), lambda i: (i, 0)),
          pl.BlockSpec((BM, D), lambda i: (i, 0)),
          pl.BlockSpec((D, H), lambda i: (0, 0)),
          pl.BlockSpec((D, H), lambda i: (0, 0)),
          pl.BlockSpec((1, H), lambda i: (0, 0)),
          pl.BlockSpec((H, H), lambda i: (0, 0)),
          pl.BlockSpec((1, H), lambda i: (0, 0)),
          pl.BlockSpec((H, 1), lambda i: (0, 0)),
          pl.BlockSpec((1, 1), lambda i: (0, 0)),
      ],
      out_specs=pl.BlockSpec((BM, 1), lambda i: (i, 0)),
      out_shape=jax.ShapeDtypeStruct((BCH, 1), jnp.float32),
  )(jd_e, cv_e, wj, wc, bc, w1, b1, w2, b2)


@jax.jit
def kernel(jd, cv, jd_table, cv_table, W_combine, b_combine, W1, b1, W2, b2):
  wj = W_combine[:D]
  wc = W_combine[D:]
  bc = b_combine.reshape(1, H)
  b1r = b1.reshape(1, H)
  b2r = b2.reshape(1, 1)
  idx_all = jnp.stack([jd, cv])
  outs = []
  for g in range(G):
    idx = (idx_all[:, g * BCH:(g + 1) * BCH]
           .reshape(2, NW, NCHUNK, CH).transpose(1, 0, 2, 3))
    jd_e, cv_e = _sc_gather(idx, jd_table, cv_table)
    outs.append(_mlp(jd_e, cv_e, wj, wc, bc, W1, b1r, W2, b2r))
  return jnp.concatenate(outs, axis=0)


# speedup vs baseline: 1.0325x; 1.0325x over previous
import functools

import jax
import jax.numpy as jnp
from jax import lax
from jax.experimental import pallas as pl
from jax.experimental.pallas import tpu as pltpu
from jax.experimental.pallas import tpu_sc as plsc

B = 4096
V = 100000
D = 1536
H = 512

NC = 2
NS = 16
NW = NC * NS
CH = 16
NBUF = 4

G = 2
BCH = B // G
ROWS_PER_W = BCH // NW
NCHUNK = ROWS_PER_W // CH

BM = 512


def _sc_gather_body(idx_hbm, jd_tab, cv_tab, jd_out, cv_out, idx_v, *rest):
  bufs = rest[:NBUF]
  gsems = rest[NBUF:2 * NBUF]
  wsems = rest[2 * NBUF:3 * NBUF]
  cid = lax.axis_index("c")
  sid = lax.axis_index("s")
  wid = sid * NC + cid
  pltpu.sync_copy(idx_hbm.at[wid], idx_v)
  base = wid * ROWS_PER_W
  work = [(t, ch) for t in range(2) for ch in range(NCHUNK)]
  tabs = (jd_tab, cv_tab)
  outs = (jd_out, cv_out)
  n = len(work)
  g = [None] * NBUF
  w = [None] * NBUF
  for k, (t, ch) in enumerate(work):
    p = k % NBUF
    if k >= NBUF:
      w[p].wait()
    g[p] = pltpu.async_copy(tabs[t].at[idx_v.at[t, ch]], bufs[p], gsems[p])
    if k >= 1:
      pt, pch = work[k - 1]
      pp = (k - 1) % NBUF
      g[pp].wait()
      w[pp] = pltpu.async_copy(
          bufs[pp], outs[pt].at[pl.ds(base + pch * CH, CH)], wsems[pp])
  lt, lch = work[n - 1]
  lp = (n - 1) % NBUF
  g[lp].wait()
  w[lp] = pltpu.async_copy(
      bufs[lp], outs[lt].at[pl.ds(base + lch * CH, CH)], wsems[lp])
  for k in range(max(0, n - NBUF), n):
    w[k % NBUF].wait()


def _sc_gather(idx, jd_table, cv_table):
  mesh = plsc.VectorSubcoreMesh(core_axis_name="c", subcore_axis_name="s")
  return pl.kernel(
      _sc_gather_body,
      mesh=mesh,
      out_type=[
          jax.ShapeDtypeStruct((BCH, D), jnp.float32),
          jax.ShapeDtypeStruct((BCH, D), jnp.float32),
      ],
      scratch_types=(
          [pltpu.VMEM((2, NCHUNK, CH), jnp.int32)]
          + [pltpu.VMEM((CH, D), jnp.float32) for _ in range(NBUF)]
          + [pltpu.SemaphoreType.DMA for _ in range(2 * NBUF)]
      ),
  )(idx, jd_table, cv_table)


def _mlp_body(jd_ref, cv_ref, wcomb_ref, bc_ref, w1_ref, b1_ref,
              w2_ref, b2_ref, out_ref):
  x = (jnp.dot(jd_ref[...], wcomb_ref[0:D], preferred_element_type=jnp.float32)
       + jnp.dot(cv_ref[...], wcomb_ref[D:2 * D],
                 preferred_element_type=jnp.float32)
       + bc_ref[...])
  x = jnp.where(x >= 0, x, 0.01 * x)
  x = jnp.dot(x, w1_ref[...], preferred_element_type=jnp.float32) + b1_ref[...]
  x = jnp.where(x >= 0, x, 0.01 * x)
  out_ref[...] = (
      jnp.dot(x, w2_ref[...], preferred_element_type=jnp.float32) + b2_ref[...])


def _mlp(jd_e, cv_e, wcomb, bc, w1, b1, w2, b2):
  return pl.pallas_call(
      _mlp_body,
      grid=(BCH // BM,),
      in_specs=[
          pl.BlockSpec((BM, D), lambda i: (i, 0)),
          pl.BlockSpec((BM, D), lambda i: (i, 0)),
          pl.BlockSpec((2 * D, H), lambda i: (0, 0)),
          pl.BlockSpec((1, H), lambda i: (0, 0)),
          pl.BlockSpec((H, H), lambda i: (0, 0)),
          pl.BlockSpec((1, H), lambda i: (0, 0)),
          pl.BlockSpec((H, 1), lambda i: (0, 0)),
          pl.BlockSpec((1, 1), lambda i: (0, 0)),
      ],
      out_specs=pl.BlockSpec((BM, 1), lambda i: (i, 0)),
      out_shape=jax.ShapeDtypeStruct((BCH, 1), jnp.float32),
  )(jd_e, cv_e, wcomb, bc, w1, b1, w2, b2)


@jax.jit
def kernel(jd, cv, jd_table, cv_table, W_combine, b_combine, W1, b1, W2, b2):
  bc = b_combine.reshape(1, H)
  b1r = b1.reshape(1, H)
  b2r = b2.reshape(1, 1)
  idx_all = jnp.stack([jd, cv])
  outs = []
  for g in range(G):
    idx = (idx_all[:, g * BCH:(g + 1) * BCH]
           .reshape(2, NW, NCHUNK, CH).transpose(1, 0, 2, 3))
    jd_e, cv_e = _sc_gather(idx, jd_table, cv_table)
    outs.append(_mlp(jd_e, cv_e, W_combine, bc, W1, b1r, W2, b2r))
  return jnp.concatenate(outs, axis=0)


# speedup vs baseline: 1.0371x; 1.0045x over previous
import functools

import jax
import jax.numpy as jnp
from jax import lax
from jax.experimental import pallas as pl
from jax.experimental.pallas import tpu as pltpu
from jax.experimental.pallas import tpu_sc as plsc

B = 4096
V = 100000
D = 1536
H = 512

NC = 2
NS = 16
NW = NC * NS
CH = 16
NBUF = 4

G = 2
BCH = B // G
ROWS_PER_W = BCH // NW
NCHUNK = ROWS_PER_W // CH

BM = 512


def _sc_gather_body(idx_hbm, jd_tab, cv_tab, jd_out, cv_out, idx_v, *rest):
  bufs = rest[:NBUF]
  gsems = rest[NBUF:2 * NBUF]
  wsems = rest[2 * NBUF:3 * NBUF]
  cid = lax.axis_index("c")
  sid = lax.axis_index("s")
  wid = sid * NC + cid
  pltpu.sync_copy(idx_hbm.at[wid], idx_v)
  base = wid * ROWS_PER_W
  work = [(t, ch) for t in range(2) for ch in range(NCHUNK)]
  tabs = (jd_tab, cv_tab)
  outs = (jd_out, cv_out)
  n = len(work)
  g = [None] * NBUF
  w = [None] * NBUF
  for k, (t, ch) in enumerate(work):
    p = k % NBUF
    if k >= NBUF:
      w[p].wait()
    g[p] = pltpu.async_copy(tabs[t].at[idx_v.at[t, ch]], bufs[p], gsems[p])
    if k >= 1:
      pt, pch = work[k - 1]
      pp = (k - 1) % NBUF
      g[pp].wait()
      w[pp] = pltpu.async_copy(
          bufs[pp], outs[pt].at[pl.ds(base + pch * CH, CH)], wsems[pp])
  lt, lch = work[n - 1]
  lp = (n - 1) % NBUF
  g[lp].wait()
  w[lp] = pltpu.async_copy(
      bufs[lp], outs[lt].at[pl.ds(base + lch * CH, CH)], wsems[lp])
  for k in range(max(0, n - NBUF), n):
    w[k % NBUF].wait()


def _sc_gather(idx, jd_table, cv_table):
  mesh = plsc.VectorSubcoreMesh(core_axis_name="c", subcore_axis_name="s")
  return pl.kernel(
      _sc_gather_body,
      mesh=mesh,
      out_type=[
          jax.ShapeDtypeStruct((BCH, D), jnp.float32),
          jax.ShapeDtypeStruct((BCH, D), jnp.float32),
      ],
      scratch_types=(
          [pltpu.VMEM((2, NCHUNK, CH), jnp.int32)]
          + [pltpu.VMEM((CH, D), jnp.float32) for _ in range(NBUF)]
          + [pltpu.SemaphoreType.DMA for _ in range(2 * NBUF)]
      ),
  )(idx, jd_table, cv_table)


def _mlp_body(jd_ref, cv_ref, wcomb_ref, bc_ref, w1_ref, b1_ref,
              w2_ref, b2_ref, out_ref):
  jd_b = jd_ref[...].astype(jnp.bfloat16)
  cv_b = cv_ref[...].astype(jnp.bfloat16)
  x = (jnp.dot(jd_b, wcomb_ref[0:D], preferred_element_type=jnp.float32)
       + jnp.dot(cv_b, wcomb_ref[D:2 * D],
                 preferred_element_type=jnp.float32)
       + bc_ref[...])
  x = jnp.where(x >= 0, x, 0.01 * x)
  x = (jnp.dot(x.astype(jnp.bfloat16), w1_ref[...],
               preferred_element_type=jnp.float32) + b1_ref[...])
  x = jnp.where(x >= 0, x, 0.01 * x)
  out_ref[...] = (
      jnp.dot(x.astype(jnp.bfloat16), w2_ref[...],
              preferred_element_type=jnp.float32) + b2_ref[...])


def _mlp(jd_e, cv_e, wcomb, bc, w1, b1, w2, b2):
  return pl.pallas_call(
      _mlp_body,
      grid=(BCH // BM,),
      in_specs=[
          pl.BlockSpec((BM, D), lambda i: (i, 0)),
          pl.BlockSpec((BM, D), lambda i: (i, 0)),
          pl.BlockSpec((2 * D, H), lambda i: (0, 0)),
          pl.BlockSpec((1, H), lambda i: (0, 0)),
          pl.BlockSpec((H, H), lambda i: (0, 0)),
          pl.BlockSpec((1, H), lambda i: (0, 0)),
          pl.BlockSpec((H, 1), lambda i: (0, 0)),
          pl.BlockSpec((1, 1), lambda i: (0, 0)),
      ],
      out_specs=pl.BlockSpec((BM, 1), lambda i: (i, 0)),
      out_shape=jax.ShapeDtypeStruct((BCH, 1), jnp.float32),
  )(jd_e, cv_e, wcomb, bc, w1, b1, w2, b2)


@jax.jit
def kernel(jd, cv, jd_table, cv_table, W_combine, b_combine, W1, b1, W2, b2):
  bc = b_combine.reshape(1, H)
  b1r = b1.reshape(1, H)
  b2r = b2.reshape(1, 1)
  wcomb = W_combine.astype(jnp.bfloat16)
  w1 = W1.astype(jnp.bfloat16)
  w2 = W2.astype(jnp.bfloat16)
  idx_all = jnp.stack([jd, cv])
  outs = []
  for g in range(G):
    idx = (idx_all[:, g * BCH:(g + 1) * BCH]
           .reshape(2, NW, NCHUNK, CH).transpose(1, 0, 2, 3))
    jd_e, cv_e = _sc_gather(idx, jd_table, cv_table)
    outs.append(_mlp(jd_e, cv_e, wcomb, bc, w1, b1r, w2, b2r))
  return jnp.concatenate(outs, axis=0)
